# baseline (device time: 121174 ns/iter reference)
import functools

import jax
import jax.numpy as jnp
from jax import lax
from jax.experimental import pallas as pl
from jax.experimental.pallas import tpu as pltpu


def kernel(x, router_W, route_idx, expert_W):
    T, D = x.shape
    E = router_W.shape[1]
    EL, _, H = expert_W.shape
    N_DEV = E // EL
    F_HOPS = N_DEV // 2 - 1
    B_HOPS = N_DEV // 2 - 1
    OPP_SLOT = N_DEV // 2

    def body(x_ref, rw_ref, idx_ref, ew_ref, out_ref,
             comm_ref, xbf_ref, g_ref,
             f_send, f_recv, b_send, b_recv, e_send, e_recv):
        my = lax.axis_index("i")
        left = lax.rem(my - 1 + N_DEV, N_DEV)
        right = lax.rem(my + 1, N_DEV)
        opp = lax.rem(my + N_DEV // 2, N_DEV)

        barrier_sem = pltpu.get_barrier_semaphore()
        for nbr in (left, right, opp):
            pl.semaphore_signal(barrier_sem, inc=1, device_id=(nbr,),
                                device_id_type=pl.DeviceIdType.MESH)
        pl.semaphore_wait(barrier_sem, 3)

        for k in range(EL):
            comm_ref[0, k] = ew_ref[k].astype(jnp.bfloat16)

        def make_fwd(h, c):
            return pltpu.make_async_remote_copy(
                src_ref=comm_ref.at[h, c],
                dst_ref=comm_ref.at[h + 1, c],
                send_sem=f_send.at[h, c],
                recv_sem=f_recv.at[h, c],
                device_id=(right,),
                device_id_type=pl.DeviceIdType.MESH,
            )

        def make_bwd(h, c):
            return pltpu.make_async_remote_copy(
                src_ref=comm_ref.at[(N_DEV - h) % N_DEV, c],
                dst_ref=comm_ref.at[N_DEV - 1 - h, c],
                send_sem=b_send.at[h, c],
                recv_sem=b_recv.at[h, c],
                device_id=(left,),
                device_id_type=pl.DeviceIdType.MESH,
            )

        rf = [[make_fwd(h, c) for c in range(EL)] for h in range(F_HOPS)]
        rb = [[make_bwd(h, c) for c in range(EL)] for h in range(B_HOPS)]
        re = [pltpu.make_async_remote_copy(
                  src_ref=comm_ref.at[0, c],
                  dst_ref=comm_ref.at[OPP_SLOT, c],
                  send_sem=e_send.at[c],
                  recv_sem=e_recv.at[c],
                  device_id=(opp,),
                  device_id_type=pl.DeviceIdType.MESH,
              ) for c in range(EL)]
        for c in range(EL):
            rf[0][c].start()
            rb[0][c].start()
            re[c].start()

        xf = x_ref[...]
        scores = jnp.dot(xf, rw_ref[...], preferred_element_type=jnp.float32)
        p = jnp.exp(scores - jnp.max(scores, axis=-1, keepdims=True))
        e_ids = lax.broadcasted_iota(jnp.int32, (T, E), 1)
        msk = (e_ids == idx_ref[:, 0:1]) | (e_ids == idx_ref[:, 1:2])
        gp = jnp.where(msk, p, 0.0)
        g_ref[...] = gp / jnp.sum(gp, axis=-1, keepdims=True)
        xbf_ref[...] = xf.astype(jnp.bfloat16)

        out_ref[...] = jnp.zeros((T, H), jnp.float32)

        def consume(slot, src):
            eb = lax.broadcasted_iota(jnp.int32, (E, EL), 0)
            kb = lax.broadcasted_iota(jnp.int32, (E, EL), 1)
            onehot = (eb == src * EL + kb).astype(jnp.float32)
            gblk = jnp.dot(g_ref[...], onehot,
                           preferred_element_type=jnp.float32)
            acc = out_ref[...]
            for k in range(EL):
                y = jnp.dot(xbf_ref[...], comm_ref[slot, k],
                            preferred_element_type=jnp.float32)
                acc = acc + gblk[:, k:k + 1] * y
            out_ref[...] = acc

        consume(0, my)
        for h in range(1, F_HOPS + 1):
            for c in range(EL):
                rf[h - 1][c].wait_recv()
                if h < F_HOPS:
                    rf[h][c].start()
            for c in range(EL):
                rb[h - 1][c].wait_recv()
                if h < B_HOPS:
                    rb[h][c].start()
            consume(h, lax.rem(my - h + N_DEV, N_DEV))
            consume(N_DEV - h, lax.rem(my + h, N_DEV))
        for c in range(EL):
            re[c].wait_recv()
        consume(OPP_SLOT, opp)

        for group in rf + rb + [re]:
            for r in group:
                r.wait_send()

        @functools.partial(pl.run_scoped, sem2=pltpu.SemaphoreType.REGULAR)
        def _(sem2):
            for nbr in (left, right, opp):
                pl.semaphore_signal(sem2, inc=1, device_id=(nbr,),
                                    device_id_type=pl.DeviceIdType.MESH)
            pl.semaphore_wait(sem2, 3)

    return pl.pallas_call(
        body,
        out_shape=jax.ShapeDtypeStruct((T, H), jnp.float32),
        in_specs=[pl.BlockSpec(memory_space=pltpu.VMEM)] * 4,
        out_specs=pl.BlockSpec(memory_space=pltpu.VMEM),
        scratch_shapes=[
            pltpu.VMEM((N_DEV, EL, D, H), jnp.bfloat16),
            pltpu.VMEM((T, D), jnp.bfloat16),
            pltpu.VMEM((T, E), jnp.float32),
            pltpu.SemaphoreType.DMA((F_HOPS, EL)),
            pltpu.SemaphoreType.DMA((F_HOPS, EL)),
            pltpu.SemaphoreType.DMA((B_HOPS, EL)),
            pltpu.SemaphoreType.DMA((B_HOPS, EL)),
            pltpu.SemaphoreType.DMA((EL,)),
            pltpu.SemaphoreType.DMA((EL,)),
        ],
        compiler_params=pltpu.CompilerParams(collective_id=0),
    )(x, router_W, route_idx, expert_W)


# device time: 106827 ns/iter; 1.1343x vs baseline; 1.1343x over previous
import functools

import jax
import jax.numpy as jnp
from jax import lax
from jax.experimental import pallas as pl
from jax.experimental.pallas import tpu as pltpu


def kernel(x, router_W, route_idx, expert_W):
    T, D = x.shape
    E = router_W.shape[1]
    EL, _, H = expert_W.shape
    N_DEV = E // EL
    F_HOPS = N_DEV // 2
    B_HOPS = N_DEV - 1 - F_HOPS

    def body(x_ref, rw_ref, idx_ref, ew_ref, out_ref,
             comm_ref, xbf_ref, g_ref,
             f_send, f_recv, b_send, b_recv):
        my = lax.axis_index("i")
        left = lax.rem(my - 1 + N_DEV, N_DEV)
        right = lax.rem(my + 1, N_DEV)

        barrier_sem = pltpu.get_barrier_semaphore()
        for nbr in (left, right):
            pl.semaphore_signal(barrier_sem, inc=1, device_id=(nbr,),
                                device_id_type=pl.DeviceIdType.MESH)
        pl.semaphore_wait(barrier_sem, 2)

        for k in range(EL):
            comm_ref[0, k] = ew_ref[k].astype(jnp.bfloat16)

        def make_fwd(h, c):
            return pltpu.make_async_remote_copy(
                src_ref=comm_ref.at[h, c],
                dst_ref=comm_ref.at[h + 1, c],
                send_sem=f_send.at[h, c],
                recv_sem=f_recv.at[h, c],
                device_id=(right,),
                device_id_type=pl.DeviceIdType.MESH,
            )

        def make_bwd(h, c):
            return pltpu.make_async_remote_copy(
                src_ref=comm_ref.at[(N_DEV - h) % N_DEV, c],
                dst_ref=comm_ref.at[N_DEV - 1 - h, c],
                send_sem=b_send.at[h, c],
                recv_sem=b_recv.at[h, c],
                device_id=(left,),
                device_id_type=pl.DeviceIdType.MESH,
            )

        rf = [[make_fwd(h, c) for c in range(EL)] for h in range(F_HOPS)]
        rb = [[make_bwd(h, c) for c in range(EL)] for h in range(B_HOPS)]
        for c in range(EL):
            rf[0][c].start()
            rb[0][c].start()

        xf = x_ref[...]
        scores = jnp.dot(xf, rw_ref[...], preferred_element_type=jnp.float32)
        p = jnp.exp(scores - jnp.max(scores, axis=-1, keepdims=True))
        e_ids = lax.broadcasted_iota(jnp.int32, (T, E), 1)
        msk = (e_ids == idx_ref[:, 0:1]) | (e_ids == idx_ref[:, 1:2])
        gp = jnp.where(msk, p, 0.0)
        g_ref[...] = gp / jnp.sum(gp, axis=-1, keepdims=True)
        xbf_ref[...] = xf.astype(jnp.bfloat16)

        out_ref[...] = jnp.zeros((T, H), jnp.float32)

        def consume(slot, src):
            eb = lax.broadcasted_iota(jnp.int32, (E, EL), 0)
            kb = lax.broadcasted_iota(jnp.int32, (E, EL), 1)
            onehot = (eb == src * EL + kb).astype(jnp.float32)
            gblk = jnp.dot(g_ref[...], onehot,
                           preferred_element_type=jnp.float32)
            acc = out_ref[...]
            for k in range(EL):
                y = jnp.dot(xbf_ref[...], comm_ref[slot, k],
                            preferred_element_type=jnp.float32)
                acc = acc + gblk[:, k:k + 1] * y
            out_ref[...] = acc

        consume(0, my)
        for h in range(1, F_HOPS + 1):
            for c in range(EL):
                rf[h - 1][c].wait_recv()
                if h < F_HOPS:
                    rf[h][c].start()
            for c in range(EL):
                if h - 1 < B_HOPS:
                    rb[h - 1][c].wait_recv()
                if h < B_HOPS:
                    rb[h][c].start()
            consume(h, lax.rem(my - h + N_DEV, N_DEV))
            if h <= B_HOPS:
                consume(N_DEV - h, lax.rem(my + h, N_DEV))
        for group in rf + rb:
            for r in group:
                r.wait_send()

        @functools.partial(pl.run_scoped, sem2=pltpu.SemaphoreType.REGULAR)
        def _(sem2):
            for nbr in (left, right):
                pl.semaphore_signal(sem2, inc=1, device_id=(nbr,),
                                    device_id_type=pl.DeviceIdType.MESH)
            pl.semaphore_wait(sem2, 2)

    return pl.pallas_call(
        body,
        out_shape=jax.ShapeDtypeStruct((T, H), jnp.float32),
        in_specs=[pl.BlockSpec(memory_space=pltpu.VMEM)] * 4,
        out_specs=pl.BlockSpec(memory_space=pltpu.VMEM),
        scratch_shapes=[
            pltpu.VMEM((N_DEV, EL, D, H), jnp.bfloat16),
            pltpu.VMEM((T, D), jnp.bfloat16),
            pltpu.VMEM((T, E), jnp.float32),
            pltpu.SemaphoreType.DMA((F_HOPS, EL)),
            pltpu.SemaphoreType.DMA((F_HOPS, EL)),
            pltpu.SemaphoreType.DMA((B_HOPS, EL)),
            pltpu.SemaphoreType.DMA((B_HOPS, EL)),
        ],
        compiler_params=pltpu.CompilerParams(collective_id=0),
    )(x, router_W, route_idx, expert_W)
